# baseline (speedup 1.0000x reference)
import functools

import jax
import jax.numpy as jnp
from jax import lax
from jax.experimental import pallas as pl
from jax.experimental.pallas import tpu as pltpu
from jax.experimental.pallas import tpu_sc as plsc

N_SPINS = 20
BATCH = 16384
NC = 2
NS = 16
NW = NC * NS
B_PER_W = BATCH // NW
LANES = 16
GROUPS = B_PER_W // LANES
CHUNK = 128
NCHUNK = B_PER_W // CHUNK

PACK_BLK = 2048

_LN2 = 0.6931471805599453
_SQRT2 = 1.4142135623730951


def _pack_body(xt_ref, idx_ref):
    p = jnp.left_shift(
        jnp.int32(1),
        N_SPINS - 1 - lax.broadcasted_iota(jnp.int32, (N_SPINS, 1), 0))
    idx_ref[...] = jnp.sum(xt_ref[...] * p, axis=0)


_pack = pl.pallas_call(
    _pack_body,
    in_specs=[pl.BlockSpec((N_SPINS, BATCH), lambda: (0, 0))],
    out_specs=pl.BlockSpec((BATCH,), lambda: (0,)),
    out_shape=jax.ShapeDtypeStruct((BATCH,), jnp.int32),
    compiler_params=pltpu.CompilerParams(disable_bounds_checks=True),
)


def _log16(v):
    bits = lax.bitcast_convert_type(v, jnp.int32)
    e = (bits >> 23) - 127
    m = lax.bitcast_convert_type(
        (bits & 0x007FFFFF) | 0x3F800000, jnp.float32)
    t = (m - 1.0) / (m + 1.0)
    t2 = t * t
    poly = 2.0 * t * (1.0 + t2 * (1.0 / 3.0 + t2 * (
        1.0 / 5.0 + t2 * (1.0 / 7.0 + t2 * (1.0 / 9.0)))))
    return e.astype(jnp.float32) * _LN2 + poly


@functools.partial(
    pl.kernel,
    out_type=jax.ShapeDtypeStruct((BATCH,), jnp.float32),
    mesh=plsc.VectorSubcoreMesh(
        core_axis_name="c", subcore_axis_name="s",
        num_cores=NC, num_subcores=NS),
    scratch_types=[
        pltpu.VMEM((B_PER_W,), jnp.int32),
        pltpu.VMEM((B_PER_W,), jnp.float32),
        pltpu.SemaphoreType.DMA,
    ],
    compiler_params=pltpu.CompilerParams(
        needs_layout_passes=False, disable_bounds_checks=True),
)
def _gather_log_sc(idx_hbm, table_hbm, out_hbm, idx_v, val_v, sem):
    wid = lax.axis_index("s") * NC + lax.axis_index("c")
    base = wid * B_PER_W

    pltpu.sync_copy(idx_hbm.at[pl.ds(base, B_PER_W)], idx_v)

    handles = [
        pltpu.async_copy(
            table_hbm.at[idx_v.at[pl.ds(k * CHUNK, CHUNK)]],
            val_v.at[pl.ds(k * CHUNK, CHUNK)], sem)
        for k in range(NCHUNK)
    ]
    for h in handles:
        h.wait()

    def log_group(g, carry):
        val_v[pl.ds(g * LANES, LANES)] = _log16(val_v[pl.ds(g * LANES, LANES)])
        return carry

    lax.fori_loop(0, GROUPS, log_group, 0)

    pltpu.sync_copy(val_v, out_hbm.at[pl.ds(base, B_PER_W)])


def kernel(x, table, j1):
    del j1
    return _gather_log_sc(_pack(jnp.swapaxes(x, 0, 1)), table)


# speedup vs baseline: 1.0258x; 1.0258x over previous
import functools

import jax
import jax.numpy as jnp
from jax import lax
from jax.experimental import pallas as pl
from jax.experimental.pallas import tpu as pltpu
from jax.experimental.pallas import tpu_sc as plsc

N_SPINS = 20
BATCH = 16384
NC = 2
NS = 16
NW = NC * NS
B_PER_W = BATCH // NW
LANES = 16
GROUPS = B_PER_W // LANES
CHUNK = 128
NCHUNK = B_PER_W // CHUNK

PACK_BLK = 2048

_LN2 = 0.6931471805599453
_SQRT2 = 1.4142135623730951


def _pack_body(xt_ref, idx_ref):
    p = jnp.left_shift(
        jnp.int32(1),
        N_SPINS - 1 - lax.broadcasted_iota(jnp.int32, (N_SPINS, 1), 0))
    idx_ref[...] = jnp.sum(xt_ref[...] * p, axis=0)


_pack = pl.pallas_call(
    _pack_body,
    in_specs=[pl.BlockSpec((N_SPINS, BATCH), lambda: (0, 0))],
    out_specs=pl.BlockSpec((BATCH,), lambda: (0,)),
    out_shape=jax.ShapeDtypeStruct((BATCH,), jnp.int32),
)


def _log16(v):
    bits = lax.bitcast_convert_type(v, jnp.int32)
    e = (bits >> 23) - 127
    m = lax.bitcast_convert_type(
        (bits & 0x007FFFFF) | 0x3F800000, jnp.float32)
    t = (m - 1.0) / (m + 1.0)
    t2 = t * t
    poly = 2.0 * t * (1.0 + t2 * (1.0 / 3.0 + t2 * (
        1.0 / 5.0 + t2 * (1.0 / 7.0 + t2 * (1.0 / 9.0)))))
    return e.astype(jnp.float32) * _LN2 + poly


@functools.partial(
    pl.kernel,
    out_type=jax.ShapeDtypeStruct((BATCH,), jnp.float32),
    mesh=plsc.VectorSubcoreMesh(
        core_axis_name="c", subcore_axis_name="s",
        num_cores=NC, num_subcores=NS),
    scratch_types=[
        pltpu.VMEM((B_PER_W,), jnp.int32),
        pltpu.VMEM((B_PER_W,), jnp.float32),
        pltpu.VMEM((B_PER_W,), jnp.float32),
        pltpu.SemaphoreType.DMA,
    ],
    compiler_params=pltpu.CompilerParams(needs_layout_passes=False),
)
def _gather_log_sc(idx_hbm, table_hbm, out_hbm, idx_v, val_v, out_v, sem):
    wid = lax.axis_index("s") * NC + lax.axis_index("c")
    base = wid * B_PER_W

    pltpu.sync_copy(idx_hbm.at[pl.ds(base, B_PER_W)], idx_v)

    handles = [
        pltpu.async_copy(
            table_hbm.at[idx_v.at[pl.ds(k * CHUNK, CHUNK)]],
            val_v.at[pl.ds(k * CHUNK, CHUNK)], sem)
        for k in range(NCHUNK)
    ]
    for h in handles:
        h.wait()

    def log_group(g, carry):
        out_v[pl.ds(g * LANES, LANES)] = _log16(val_v[pl.ds(g * LANES, LANES)])
        return carry

    lax.fori_loop(0, GROUPS, log_group, 0)

    pltpu.sync_copy(out_v, out_hbm.at[pl.ds(base, B_PER_W)])


def kernel(x, table, j1):
    del j1
    return _gather_log_sc(_pack(jnp.swapaxes(x, 0, 1)), table)
